# baseline (speedup 1.0000x reference)
import dataclasses
import functools

import jax
import jax.numpy as jnp
from jax import lax
from jax.experimental import pallas as pl
from jax.experimental.pallas import tpu as pltpu
from jax.experimental.pallas import tpu_sc as plsc

_N = 10000
_E = 160000
_D = 256
_ROWS = 1000
_NBLK = _N // _ROWS
_ECH = 2000
_NECH = _E // _ECH

_NW = 32
_RPT = 320
_NPAD = _NW * _RPT
_CAP = 128
_SECH = 2000
_GRP = _SECH // 16
_NBATCH = 1440
_ESLOT = _NBATCH * _CAP
_DW = _D // 2


def _linear(x, w, b, relu, out_dtype=jnp.float32):
    def body(x_ref, w_ref, b_ref, o_ref):
        y = jnp.dot(x_ref[...], w_ref[...],
                    preferred_element_type=jnp.float32) + b_ref[...]
        if relu:
            y = jnp.maximum(y, 0.0)
        o_ref[...] = y.astype(out_dtype)

    return pl.pallas_call(
        body,
        grid=(_NBLK,),
        in_specs=[
            pl.BlockSpec((_ROWS, _D), lambda i: (i, 0)),
            pl.BlockSpec((_D, _D), lambda i: (0, 0)),
            pl.BlockSpec((1, _D), lambda i: (0, 0)),
        ],
        out_specs=pl.BlockSpec((_ROWS, _D), lambda i: (i, 0)),
        out_shape=jax.ShapeDtypeStruct((_N, _D), out_dtype),
    )(x, w, b.reshape(1, _D))


def _zero_agg(aggbuf):
    zf = jnp.zeros((16,), jnp.int32)

    @pl.loop(0, _RPT + 1)
    def _(r):
        for c in range(0, _DW, 16):
            aggbuf[r, pl.ds(c, 16)] = zf


def _accum_rows(aggbuf, rowbuf, bdst):
    lane = lax.iota(jnp.int32, 16)

    @pl.loop(0, _CAP // 16)
    def _(g):
        dgrp = bdst[pl.ds(g * 16, 16)]

        @pl.loop(0, 16)
        def _(l):
            d = jnp.max(jnp.where(lane == l, dgrp, 0))
            i = g * 16 + l
            for c in range(0, _DW, 16):
                a = plsc.bitcast(aggbuf[d, pl.ds(c, 16)], jnp.bfloat16)
                v = plsc.bitcast(rowbuf[i, pl.ds(c, 16)], jnp.bfloat16)
                m = jnp.maximum(a, v)
                aggbuf[d, pl.ds(c, 16)] = plsc.bitcast(m, jnp.int32)


def _accumulate(t_hbm, aggbuf, rowbuf, bsrc, bdst, sem):
    pltpu.async_copy(t_hbm.at[bsrc], rowbuf, sem).wait()
    _accum_rows(aggbuf, rowbuf, bdst)


def _segmax_scan_body(t_hbm, src_hbm, dst_hbm,
                      out_hbm, lsrc_hbm, ldst_hbm, cnts_hbm,
                      aggbuf, rowbuf, bsrc, bdst, esrc, edst, cbuf, sem):
    wid = lax.axis_index("c") * 16 + lax.axis_index("s")
    lo = wid * _RPT

    _zero_agg(aggbuf)

    @pl.loop(0, _CAP, step=16)
    def _(i):
        bsrc[pl.ds(i, 16)] = jnp.zeros((16,), jnp.int32)
        bdst[pl.ds(i, 16)] = jnp.full((16,), _RPT, jnp.int32)

    def flush(nb):
        pltpu.sync_copy(bsrc, lsrc_hbm.at[wid, pl.ds(nb * _CAP, _CAP)])
        pltpu.sync_copy(bdst, ldst_hbm.at[wid, pl.ds(nb * _CAP, _CAP)])
        _accumulate(t_hbm, aggbuf, rowbuf, bsrc, bdst, sem)
        return nb + 1

    def group(g, carry):
        cnt, nb = carry
        svec = esrc[pl.ds(g * 16, 16)]
        dvec = edst[pl.ds(g * 16, 16)]
        mask = (dvec >= lo) & (dvec < lo + _RPT)
        npop = jnp.sum(jnp.where(mask, 1, 0))

        def do_flush():
            return 0, flush(nb)

        cnt, nb = lax.cond(cnt > _CAP - 16, do_flush, lambda: (cnt, nb))
        plsc.store_compressed(bsrc.at[pl.ds(cnt, 16)], svec, mask=mask)
        plsc.store_compressed(bdst.at[pl.ds(cnt, 16)], dvec - lo, mask=mask)
        return cnt + npop, nb

    def chunk(ch, carry):
        pltpu.sync_copy(src_hbm.at[pl.ds(ch * _SECH, _SECH)], esrc)
        pltpu.sync_copy(dst_hbm.at[pl.ds(ch * _SECH, _SECH)], edst)
        return lax.fori_loop(0, _GRP, group, carry)

    _, nb = lax.fori_loop(0, _E // _SECH, chunk, (0, 0))
    nb = flush(nb)

    cbuf[...] = jnp.full((16,), nb, jnp.int32)
    pltpu.sync_copy(cbuf, cnts_hbm.at[wid])
    pltpu.sync_copy(aggbuf.at[pl.ds(0, _RPT)], out_hbm.at[pl.ds(lo, _RPT)])


def _segmax_replay_body(t_hbm, lsrc_hbm, ldst_hbm, cnts_hbm, out_hbm,
                        aggbuf, rowA, rowB, srcA, srcB, dstA, dstB, cbuf,
                        semA, semB):
    wid = lax.axis_index("c") * 16 + lax.axis_index("s")
    lo = wid * _RPT

    _zero_agg(aggbuf)

    pltpu.sync_copy(cnts_hbm.at[wid], cbuf)
    nb = jnp.max(cbuf[...])

    def start(b, sbuf, dbuf, rbuf, sem):
        pltpu.sync_copy(lsrc_hbm.at[wid, pl.ds(b * _CAP, _CAP)], sbuf)
        pltpu.sync_copy(ldst_hbm.at[wid, pl.ds(b * _CAP, _CAP)], dbuf)
        pltpu.async_copy(t_hbm.at[sbuf], rbuf, sem)

    def finish(sbuf, dbuf, rbuf, sem):
        pltpu.make_async_copy(t_hbm.at[sbuf], rbuf, sem).wait()
        _accum_rows(aggbuf, rbuf, dbuf)

    start(0, srcA, dstA, rowA, semA)

    def pair(p, carry):
        b1 = 2 * p + 1

        @pl.when(b1 < nb)
        def _():
            start(b1, srcB, dstB, rowB, semB)

        finish(srcA, dstA, rowA, semA)

        @pl.when(b1 < nb)
        def _():
            @pl.when(b1 + 1 < nb)
            def _():
                start(b1 + 1, srcA, dstA, rowA, semA)

            finish(srcB, dstB, rowB, semB)

        return carry

    lax.fori_loop(0, (nb + 1) // 2, pair, 0)

    pltpu.sync_copy(aggbuf.at[pl.ds(0, _RPT)], out_hbm.at[pl.ds(lo, _RPT)])


def _sc_compiler_params():
    cp = pltpu.CompilerParams()
    if "needs_layout_passes" in pltpu.CompilerParams.__dataclass_fields__:
        cp = dataclasses.replace(cp, needs_layout_passes=False)
    return cp


def _segmax_scan(t, src, dst):
    k = pl.kernel(
        _segmax_scan_body,
        out_type=(
            jax.ShapeDtypeStruct((_NPAD, _DW), jnp.int32),
            jax.ShapeDtypeStruct((_NW, _ESLOT), jnp.int32),
            jax.ShapeDtypeStruct((_NW, _ESLOT), jnp.int32),
            jax.ShapeDtypeStruct((_NW, 16), jnp.int32),
        ),
        compiler_params=_sc_compiler_params(),
        mesh=plsc.VectorSubcoreMesh(core_axis_name="c", subcore_axis_name="s"),
        scratch_types=[
            pltpu.VMEM((_RPT + 1, _DW), jnp.int32),
            pltpu.VMEM((_CAP, _DW), jnp.int32),
            pltpu.VMEM((_CAP,), jnp.int32),
            pltpu.VMEM((_CAP,), jnp.int32),
            pltpu.VMEM((_SECH,), jnp.int32),
            pltpu.VMEM((_SECH,), jnp.int32),
            pltpu.VMEM((16,), jnp.int32),
            pltpu.SemaphoreType.DMA,
        ],
    )
    return k(t, src, dst)


def _segmax_replay(t, lsrc, ldst, cnts):
    k = pl.kernel(
        _segmax_replay_body,
        out_type=jax.ShapeDtypeStruct((_NPAD, _DW), jnp.int32),
        compiler_params=_sc_compiler_params(),
        mesh=plsc.VectorSubcoreMesh(core_axis_name="c", subcore_axis_name="s"),
        scratch_types=[
            pltpu.VMEM((_RPT + 1, _DW), jnp.int32),
            pltpu.VMEM((_CAP, _DW), jnp.int32),
            pltpu.VMEM((_CAP, _DW), jnp.int32),
            pltpu.VMEM((_CAP,), jnp.int32),
            pltpu.VMEM((_CAP,), jnp.int32),
            pltpu.VMEM((_CAP,), jnp.int32),
            pltpu.VMEM((_CAP,), jnp.int32),
            pltpu.VMEM((16,), jnp.int32),
            pltpu.SemaphoreType.DMA,
            pltpu.SemaphoreType.DMA,
        ],
    )
    return k(t, lsrc, ldst, cnts)


def _layer0_update(f0, agg0, w_bot):
    def body(f0_ref, agg_ref, w_ref, u_ref, ps_ref, pss_ref):
        y = f0_ref[...] + jnp.dot(agg_ref[...].astype(jnp.float32),
                                  w_ref[...],
                                  preferred_element_type=jnp.float32)
        u = jnp.maximum(y, 0.0)
        u_ref[...] = u
        ps_ref[0, ...] = jnp.sum(u, axis=0, keepdims=True)
        pss_ref[0, ...] = jnp.sum(u * u, axis=0, keepdims=True)

    return pl.pallas_call(
        body,
        grid=(_NBLK,),
        in_specs=[
            pl.BlockSpec((_ROWS, _D), lambda i: (i, 0)),
            pl.BlockSpec((_ROWS, _D), lambda i: (i, 0)),
            pl.BlockSpec((_D, _D), lambda i: (0, 0)),
        ],
        out_specs=[
            pl.BlockSpec((_ROWS, _D), lambda i: (i, 0)),
            pl.BlockSpec((1, 1, _D), lambda i: (i, 0, 0)),
            pl.BlockSpec((1, 1, _D), lambda i: (i, 0, 0)),
        ],
        out_shape=[
            jax.ShapeDtypeStruct((_N, _D), jnp.float32),
            jax.ShapeDtypeStruct((_NBLK, 1, _D), jnp.float32),
            jax.ShapeDtypeStruct((_NBLK, 1, _D), jnp.float32),
        ],
    )(f0, agg0, w_bot)


def _bn_norm_t1(u, ps, pss, gamma, beta, Wp1, bp1):
    def body(u_ref, ps_ref, pss_ref, g_ref, be_ref, w_ref, b_ref,
             h_ref, t1_ref):
        mean = jnp.sum(ps_ref[...], axis=0) / _N
        var = jnp.sum(pss_ref[...], axis=0) / _N - mean * mean
        inv = jax.lax.rsqrt(var + 1e-5)
        hb = (u_ref[...] - mean) * inv * g_ref[...] + be_ref[...]
        norm = jnp.sqrt(jnp.sum(hb * hb, axis=1, keepdims=True))
        hn = hb / (norm + 1e-6)
        h_ref[...] = hn
        t1_ref[...] = jnp.maximum(
            jnp.dot(hn, w_ref[...], preferred_element_type=jnp.float32)
            + b_ref[...], 0.0).astype(jnp.bfloat16)

    return pl.pallas_call(
        body,
        grid=(_NBLK,),
        in_specs=[
            pl.BlockSpec((_ROWS, _D), lambda i: (i, 0)),
            pl.BlockSpec((_NBLK, 1, _D), lambda i: (0, 0, 0)),
            pl.BlockSpec((_NBLK, 1, _D), lambda i: (0, 0, 0)),
            pl.BlockSpec((1, _D), lambda i: (0, 0)),
            pl.BlockSpec((1, _D), lambda i: (0, 0)),
            pl.BlockSpec((_D, _D), lambda i: (0, 0)),
            pl.BlockSpec((1, _D), lambda i: (0, 0)),
        ],
        out_specs=[
            pl.BlockSpec((_ROWS, _D), lambda i: (i, 0)),
            pl.BlockSpec((_ROWS, _D), lambda i: (i, 0)),
        ],
        out_shape=[
            jax.ShapeDtypeStruct((_N, _D), jnp.float32),
            jax.ShapeDtypeStruct((_N, _D), jnp.bfloat16),
        ],
    )(u, ps, pss, gamma.reshape(1, _D), beta.reshape(1, _D), Wp1,
      bp1.reshape(1, _D))


def _final(h1, agg1, w_bot):
    def body(h1_ref, agg_ref, w_ref, o_ref):
        o_ref[...] = h1_ref[...] + jnp.dot(
            agg_ref[...].astype(jnp.float32), w_ref[...],
            preferred_element_type=jnp.float32)

    return pl.pallas_call(
        body,
        grid=(_NBLK,),
        in_specs=[
            pl.BlockSpec((_ROWS, _D), lambda i: (i, 0)),
            pl.BlockSpec((_ROWS, _D), lambda i: (i, 0)),
            pl.BlockSpec((_D, _D), lambda i: (0, 0)),
        ],
        out_specs=pl.BlockSpec((_ROWS, _D), lambda i: (i, 0)),
        out_shape=jax.ShapeDtypeStruct((_N, _D), jnp.float32),
    )(h1, agg1, w_bot)


def _pack(t):
    return lax.bitcast_convert_type(t.reshape(_N, _DW, 2), jnp.int32)


def _unpack(a):
    return lax.bitcast_convert_type(a, jnp.bfloat16).reshape(-1, _D)


def kernel(features, edge_index, Wp0, bp0, Wp1, bp1, Wfc0, bfc0, Wfc1, bfc1,
           gamma0, beta0):
    src = edge_index[0]
    dst = edge_index[1]
    t0 = _linear(features, Wp0, bp0, relu=True, out_dtype=jnp.bfloat16)
    f0 = _linear(features, Wfc0[:_D], bfc0, relu=False)
    agg0, lsrc, ldst, cnts = _segmax_scan(_pack(t0), src, dst)
    agg0 = _unpack(agg0)[:_N]
    u, ps, pss = _layer0_update(f0, agg0, Wfc0[_D:])
    h, t1 = _bn_norm_t1(u, ps, pss, gamma0, beta0, Wp1, bp1)
    h1 = _linear(h, Wfc1[:_D], bfc1, relu=False)
    agg1 = _unpack(_segmax_replay(_pack(t1), lsrc, ldst, cnts))[:_N]
    out = _final(h1, agg1, Wfc1[_D:])
    return out


# speedup vs baseline: 1.6087x; 1.6087x over previous
import dataclasses
import functools

import jax
import jax.numpy as jnp
from jax import lax
from jax.experimental import pallas as pl
from jax.experimental.pallas import tpu as pltpu
from jax.experimental.pallas import tpu_sc as plsc

_N = 10000
_E = 160000
_D = 256
_ROWS = 1000
_NBLK = _N // _ROWS
_ECH = 2000
_NECH = _E // _ECH

_NW = 32
_RPT = 320
_NPAD = _NW * _RPT
_CAP = 512
_NGB = _CAP // 128
_SECH = 2000
_GRP = _SECH // 16
_NBATCH = 336
_ESLOT = _NBATCH * _CAP
_DW = _D // 2


def _linear(x, w, b, relu, out_dtype=jnp.float32):
    def body(x_ref, w_ref, b_ref, o_ref):
        y = jnp.dot(x_ref[...], w_ref[...],
                    preferred_element_type=jnp.float32) + b_ref[...]
        if relu:
            y = jnp.maximum(y, 0.0)
        o_ref[...] = y.astype(out_dtype)

    return pl.pallas_call(
        body,
        grid=(_NBLK,),
        in_specs=[
            pl.BlockSpec((_ROWS, _D), lambda i: (i, 0)),
            pl.BlockSpec((_D, _D), lambda i: (0, 0)),
            pl.BlockSpec((1, _D), lambda i: (0, 0)),
        ],
        out_specs=pl.BlockSpec((_ROWS, _D), lambda i: (i, 0)),
        out_shape=jax.ShapeDtypeStruct((_N, _D), out_dtype),
    )(x, w, b.reshape(1, _D))


def _zero_agg(aggbuf):
    zf = jnp.zeros((16,), jnp.int32)

    @pl.loop(0, _RPT + 1)
    def _(r):
        for c in range(0, _DW, 16):
            aggbuf[r, pl.ds(c, 16)] = zf


def _accum_rows(aggbuf, rowbuf, bdst):
    lane = lax.iota(jnp.int32, 16)

    @pl.loop(0, _CAP // 16)
    def _(g):
        dgrp = bdst[pl.ds(g * 16, 16)]

        @pl.loop(0, 16)
        def _(l):
            d = jnp.max(jnp.where(lane == l, dgrp, 0))
            i = g * 16 + l
            for c in range(0, _DW, 16):
                a = plsc.bitcast(aggbuf[d, pl.ds(c, 16)], jnp.bfloat16)
                v = plsc.bitcast(rowbuf[i, pl.ds(c, 16)], jnp.bfloat16)
                m = jnp.maximum(a, v)
                aggbuf[d, pl.ds(c, 16)] = plsc.bitcast(m, jnp.int32)


def _accumulate(t_hbm, aggbuf, rowbuf, bsrc, bdst, sem):
    for q in range(_NGB):
        pltpu.async_copy(t_hbm.at[bsrc.at[pl.ds(q * 128, 128)]],
                         rowbuf.at[pl.ds(q * 128, 128)], sem)
    for q in range(_NGB):
        pltpu.make_async_copy(t_hbm.at[bsrc.at[pl.ds(q * 128, 128)]],
                              rowbuf.at[pl.ds(q * 128, 128)], sem).wait()
    _accum_rows(aggbuf, rowbuf, bdst)


def _segmax_scan_body(t_hbm, src_hbm, dst_hbm,
                      out_hbm, lsrc_hbm, ldst_hbm, cnts_hbm,
                      aggbuf, rowbuf, bsrc, bdst, esrc, edst, cbuf, sem):
    wid = lax.axis_index("c") * 16 + lax.axis_index("s")
    lo = wid * _RPT

    _zero_agg(aggbuf)

    @pl.loop(0, _CAP, step=16)
    def _(i):
        bsrc[pl.ds(i, 16)] = jnp.zeros((16,), jnp.int32)
        bdst[pl.ds(i, 16)] = jnp.full((16,), _RPT, jnp.int32)

    def flush(nb):
        pltpu.sync_copy(bsrc, lsrc_hbm.at[wid, pl.ds(nb * _CAP, _CAP)])
        pltpu.sync_copy(bdst, ldst_hbm.at[wid, pl.ds(nb * _CAP, _CAP)])
        _accumulate(t_hbm, aggbuf, rowbuf, bsrc, bdst, sem)
        return nb + 1

    def group(g, carry):
        cnt, nb = carry
        svec = esrc[pl.ds(g * 16, 16)]
        dvec = edst[pl.ds(g * 16, 16)]
        mask = (dvec >= lo) & (dvec < lo + _RPT)
        npop = jnp.sum(jnp.where(mask, 1, 0))

        def do_flush():
            return 0, flush(nb)

        cnt, nb = lax.cond(cnt > _CAP - 16, do_flush, lambda: (cnt, nb))
        plsc.store_compressed(bsrc.at[pl.ds(cnt, 16)], svec, mask=mask)
        plsc.store_compressed(bdst.at[pl.ds(cnt, 16)], dvec - lo, mask=mask)
        return cnt + npop, nb

    def chunk(ch, carry):
        pltpu.sync_copy(src_hbm.at[pl.ds(ch * _SECH, _SECH)], esrc)
        pltpu.sync_copy(dst_hbm.at[pl.ds(ch * _SECH, _SECH)], edst)
        return lax.fori_loop(0, _GRP, group, carry)

    _, nb = lax.fori_loop(0, _E // _SECH, chunk, (0, 0))
    nb = flush(nb)

    cbuf[...] = jnp.full((16,), nb, jnp.int32)
    pltpu.sync_copy(cbuf, cnts_hbm.at[wid])
    pltpu.sync_copy(aggbuf.at[pl.ds(0, _RPT)], out_hbm.at[pl.ds(lo, _RPT)])


def _segmax_replay_body(t_hbm, lsrc_hbm, ldst_hbm, cnts_hbm, out_hbm,
                        aggbuf, rowbuf, bsrc, bdst, cbuf, sem):
    wid = lax.axis_index("c") * 16 + lax.axis_index("s")
    lo = wid * _RPT

    _zero_agg(aggbuf)

    pltpu.sync_copy(cnts_hbm.at[wid], cbuf)
    nb = jnp.max(cbuf[...])

    def batch(b, carry):
        pltpu.sync_copy(lsrc_hbm.at[wid, pl.ds(b * _CAP, _CAP)], bsrc)
        pltpu.sync_copy(ldst_hbm.at[wid, pl.ds(b * _CAP, _CAP)], bdst)
        _accumulate(t_hbm, aggbuf, rowbuf, bsrc, bdst, sem)
        return carry

    lax.fori_loop(0, nb, batch, 0)

    pltpu.sync_copy(aggbuf.at[pl.ds(0, _RPT)], out_hbm.at[pl.ds(lo, _RPT)])


def _sc_compiler_params():
    cp = pltpu.CompilerParams()
    if "needs_layout_passes" in pltpu.CompilerParams.__dataclass_fields__:
        cp = dataclasses.replace(cp, needs_layout_passes=False)
    return cp


def _segmax_scan(t, src, dst):
    k = pl.kernel(
        _segmax_scan_body,
        out_type=(
            jax.ShapeDtypeStruct((_NPAD, _DW), jnp.int32),
            jax.ShapeDtypeStruct((_NW, _ESLOT), jnp.int32),
            jax.ShapeDtypeStruct((_NW, _ESLOT), jnp.int32),
            jax.ShapeDtypeStruct((_NW, 16), jnp.int32),
        ),
        compiler_params=_sc_compiler_params(),
        mesh=plsc.VectorSubcoreMesh(core_axis_name="c", subcore_axis_name="s"),
        scratch_types=[
            pltpu.VMEM((_RPT + 1, _DW), jnp.int32),
            pltpu.VMEM((_CAP, _DW), jnp.int32),
            pltpu.VMEM((_CAP,), jnp.int32),
            pltpu.VMEM((_CAP,), jnp.int32),
            pltpu.VMEM((_SECH,), jnp.int32),
            pltpu.VMEM((_SECH,), jnp.int32),
            pltpu.VMEM((16,), jnp.int32),
            pltpu.SemaphoreType.DMA,
        ],
    )
    return k(t, src, dst)


def _segmax_replay(t, lsrc, ldst, cnts):
    k = pl.kernel(
        _segmax_replay_body,
        out_type=jax.ShapeDtypeStruct((_NPAD, _DW), jnp.int32),
        compiler_params=_sc_compiler_params(),
        mesh=plsc.VectorSubcoreMesh(core_axis_name="c", subcore_axis_name="s"),
        scratch_types=[
            pltpu.VMEM((_RPT + 1, _DW), jnp.int32),
            pltpu.VMEM((_CAP, _DW), jnp.int32),
            pltpu.VMEM((_CAP,), jnp.int32),
            pltpu.VMEM((_CAP,), jnp.int32),
            pltpu.VMEM((16,), jnp.int32),
            pltpu.SemaphoreType.DMA,
        ],
    )
    return k(t, lsrc, ldst, cnts)


def _layer0_update(f0, agg0, w_bot):
    def body(f0_ref, agg_ref, w_ref, u_ref, ps_ref, pss_ref):
        y = f0_ref[...] + jnp.dot(agg_ref[...].astype(jnp.float32),
                                  w_ref[...],
                                  preferred_element_type=jnp.float32)
        u = jnp.maximum(y, 0.0)
        u_ref[...] = u
        ps_ref[0, ...] = jnp.sum(u, axis=0, keepdims=True)
        pss_ref[0, ...] = jnp.sum(u * u, axis=0, keepdims=True)

    return pl.pallas_call(
        body,
        grid=(_NBLK,),
        in_specs=[
            pl.BlockSpec((_ROWS, _D), lambda i: (i, 0)),
            pl.BlockSpec((_ROWS, _D), lambda i: (i, 0)),
            pl.BlockSpec((_D, _D), lambda i: (0, 0)),
        ],
        out_specs=[
            pl.BlockSpec((_ROWS, _D), lambda i: (i, 0)),
            pl.BlockSpec((1, 1, _D), lambda i: (i, 0, 0)),
            pl.BlockSpec((1, 1, _D), lambda i: (i, 0, 0)),
        ],
        out_shape=[
            jax.ShapeDtypeStruct((_N, _D), jnp.float32),
            jax.ShapeDtypeStruct((_NBLK, 1, _D), jnp.float32),
            jax.ShapeDtypeStruct((_NBLK, 1, _D), jnp.float32),
        ],
    )(f0, agg0, w_bot)


def _bn_norm_t1(u, ps, pss, gamma, beta, Wp1, bp1):
    def body(u_ref, ps_ref, pss_ref, g_ref, be_ref, w_ref, b_ref,
             h_ref, t1_ref):
        mean = jnp.sum(ps_ref[...], axis=0) / _N
        var = jnp.sum(pss_ref[...], axis=0) / _N - mean * mean
        inv = jax.lax.rsqrt(var + 1e-5)
        hb = (u_ref[...] - mean) * inv * g_ref[...] + be_ref[...]
        norm = jnp.sqrt(jnp.sum(hb * hb, axis=1, keepdims=True))
        hn = hb / (norm + 1e-6)
        h_ref[...] = hn
        t1_ref[...] = jnp.maximum(
            jnp.dot(hn, w_ref[...], preferred_element_type=jnp.float32)
            + b_ref[...], 0.0).astype(jnp.bfloat16)

    return pl.pallas_call(
        body,
        grid=(_NBLK,),
        in_specs=[
            pl.BlockSpec((_ROWS, _D), lambda i: (i, 0)),
            pl.BlockSpec((_NBLK, 1, _D), lambda i: (0, 0, 0)),
            pl.BlockSpec((_NBLK, 1, _D), lambda i: (0, 0, 0)),
            pl.BlockSpec((1, _D), lambda i: (0, 0)),
            pl.BlockSpec((1, _D), lambda i: (0, 0)),
            pl.BlockSpec((_D, _D), lambda i: (0, 0)),
            pl.BlockSpec((1, _D), lambda i: (0, 0)),
        ],
        out_specs=[
            pl.BlockSpec((_ROWS, _D), lambda i: (i, 0)),
            pl.BlockSpec((_ROWS, _D), lambda i: (i, 0)),
        ],
        out_shape=[
            jax.ShapeDtypeStruct((_N, _D), jnp.float32),
            jax.ShapeDtypeStruct((_N, _D), jnp.bfloat16),
        ],
    )(u, ps, pss, gamma.reshape(1, _D), beta.reshape(1, _D), Wp1,
      bp1.reshape(1, _D))


def _final(h1, agg1, w_bot):
    def body(h1_ref, agg_ref, w_ref, o_ref):
        o_ref[...] = h1_ref[...] + jnp.dot(
            agg_ref[...].astype(jnp.float32), w_ref[...],
            preferred_element_type=jnp.float32)

    return pl.pallas_call(
        body,
        grid=(_NBLK,),
        in_specs=[
            pl.BlockSpec((_ROWS, _D), lambda i: (i, 0)),
            pl.BlockSpec((_ROWS, _D), lambda i: (i, 0)),
            pl.BlockSpec((_D, _D), lambda i: (0, 0)),
        ],
        out_specs=pl.BlockSpec((_ROWS, _D), lambda i: (i, 0)),
        out_shape=jax.ShapeDtypeStruct((_N, _D), jnp.float32),
    )(h1, agg1, w_bot)


def _pack(t):
    return lax.bitcast_convert_type(t.reshape(_N, _DW, 2), jnp.int32)


def _unpack(a):
    return lax.bitcast_convert_type(a, jnp.bfloat16).reshape(-1, _D)


def kernel(features, edge_index, Wp0, bp0, Wp1, bp1, Wfc0, bfc0, Wfc1, bfc1,
           gamma0, beta0):
    src = edge_index[0]
    dst = edge_index[1]
    t0 = _linear(features, Wp0, bp0, relu=True, out_dtype=jnp.bfloat16)
    f0 = _linear(features, Wfc0[:_D], bfc0, relu=False)
    agg0, lsrc, ldst, cnts = _segmax_scan(_pack(t0), src, dst)
    agg0 = _unpack(agg0)[:_N]
    u, ps, pss = _layer0_update(f0, agg0, Wfc0[_D:])
    h, t1 = _bn_norm_t1(u, ps, pss, gamma0, beta0, Wp1, bp1)
    h1 = _linear(h, Wfc1[:_D], bfc1, relu=False)
    agg1 = _unpack(_segmax_replay(_pack(t1), lsrc, ldst, cnts))[:_N]
    out = _final(h1, agg1, Wfc1[_D:])
    return out


# speedup vs baseline: 1.6976x; 1.0553x over previous
import dataclasses
import functools

import jax
import jax.numpy as jnp
from jax import lax
from jax.experimental import pallas as pl
from jax.experimental.pallas import tpu as pltpu
from jax.experimental.pallas import tpu_sc as plsc

_N = 10000
_E = 160000
_D = 256
_ROWS = 1000
_NBLK = _N // _ROWS
_ECH = 2000
_NECH = _E // _ECH

_NW = 32
_RPT = 320
_NPAD = _NW * _RPT
_CAP = 512
_NGB = _CAP // 128
_SECH = 8000
_GRP = _SECH // 16
_NBATCH = 336
_ESLOT = _NBATCH * _CAP
_DW = _D // 2


def _linear(x, w, b, relu, out_dtype=jnp.float32):
    def body(x_ref, w_ref, b_ref, o_ref):
        y = jnp.dot(x_ref[...], w_ref[...],
                    preferred_element_type=jnp.float32) + b_ref[...]
        if relu:
            y = jnp.maximum(y, 0.0)
        o_ref[...] = y.astype(out_dtype)

    return pl.pallas_call(
        body,
        grid=(_NBLK,),
        in_specs=[
            pl.BlockSpec((_ROWS, _D), lambda i: (i, 0)),
            pl.BlockSpec((_D, _D), lambda i: (0, 0)),
            pl.BlockSpec((1, _D), lambda i: (0, 0)),
        ],
        out_specs=pl.BlockSpec((_ROWS, _D), lambda i: (i, 0)),
        out_shape=jax.ShapeDtypeStruct((_N, _D), out_dtype),
    )(x, w, b.reshape(1, _D))


def _zero_agg(aggbuf):
    zf = jnp.zeros((16,), jnp.int32)

    @pl.loop(0, _RPT + 1)
    def _(r):
        for c in range(0, _DW, 16):
            aggbuf[r, pl.ds(c, 16)] = zf


def _accum_rows(aggbuf, rowbuf, bdst):
    lane = lax.iota(jnp.int32, 16)

    @pl.loop(0, _CAP // 16)
    def _(g):
        dgrp = bdst[pl.ds(g * 16, 16)]

        @pl.loop(0, 16)
        def _(l):
            d = jnp.max(jnp.where(lane == l, dgrp, 0))
            i = g * 16 + l
            for c in range(0, _DW, 16):
                a = plsc.bitcast(aggbuf[d, pl.ds(c, 16)], jnp.bfloat16)
                v = plsc.bitcast(rowbuf[i, pl.ds(c, 16)], jnp.bfloat16)
                m = jnp.maximum(a, v)
                aggbuf[d, pl.ds(c, 16)] = plsc.bitcast(m, jnp.int32)


def _accumulate(t_hbm, aggbuf, rowbuf, bsrc, bdst, sem):
    for q in range(_NGB):
        pltpu.async_copy(t_hbm.at[bsrc.at[pl.ds(q * 128, 128)]],
                         rowbuf.at[pl.ds(q * 128, 128)], sem)
    for q in range(_NGB):
        pltpu.make_async_copy(t_hbm.at[bsrc.at[pl.ds(q * 128, 128)]],
                              rowbuf.at[pl.ds(q * 128, 128)], sem).wait()
    _accum_rows(aggbuf, rowbuf, bdst)


def _segmax_scan_body(t_hbm, src_hbm, dst_hbm,
                      out_hbm, lsrc_hbm, ldst_hbm, cnts_hbm,
                      aggbuf, rowbuf, bsrc, bdst, esrc, edst, cbuf, sem):
    wid = lax.axis_index("c") * 16 + lax.axis_index("s")
    lo = wid * _RPT

    _zero_agg(aggbuf)

    @pl.loop(0, _CAP, step=16)
    def _(i):
        bsrc[pl.ds(i, 16)] = jnp.zeros((16,), jnp.int32)
        bdst[pl.ds(i, 16)] = jnp.full((16,), _RPT, jnp.int32)

    def flush(nb):
        pltpu.sync_copy(bsrc, lsrc_hbm.at[wid, pl.ds(nb * _CAP, _CAP)])
        pltpu.sync_copy(bdst, ldst_hbm.at[wid, pl.ds(nb * _CAP, _CAP)])
        _accumulate(t_hbm, aggbuf, rowbuf, bsrc, bdst, sem)
        return nb + 1

    def group(g, carry):
        cnt, nb = carry
        svec = esrc[pl.ds(g * 16, 16)]
        dvec = edst[pl.ds(g * 16, 16)]
        mask = (dvec >= lo) & (dvec < lo + _RPT)
        npop = jnp.sum(jnp.where(mask, 1, 0))

        def do_flush():
            return 0, flush(nb)

        cnt, nb = lax.cond(cnt > _CAP - 16, do_flush, lambda: (cnt, nb))
        plsc.store_compressed(bsrc.at[pl.ds(cnt, 16)], svec, mask=mask)
        plsc.store_compressed(bdst.at[pl.ds(cnt, 16)], dvec - lo, mask=mask)
        return cnt + npop, nb

    def chunk(ch, carry):
        pltpu.sync_copy(src_hbm.at[pl.ds(ch * _SECH, _SECH)], esrc)
        pltpu.sync_copy(dst_hbm.at[pl.ds(ch * _SECH, _SECH)], edst)
        return lax.fori_loop(0, _GRP, group, carry)

    _, nb = lax.fori_loop(0, _E // _SECH, chunk, (0, 0))
    nb = flush(nb)

    cbuf[...] = jnp.full((16,), nb, jnp.int32)
    pltpu.sync_copy(cbuf, cnts_hbm.at[wid])
    pltpu.sync_copy(aggbuf.at[pl.ds(0, _RPT)], out_hbm.at[pl.ds(lo, _RPT)])


def _segmax_replay_body(t_hbm, lsrc_hbm, ldst_hbm, cnts_hbm, out_hbm,
                        aggbuf, rowbuf, bsrc, bdst, cbuf, sem):
    wid = lax.axis_index("c") * 16 + lax.axis_index("s")
    lo = wid * _RPT

    _zero_agg(aggbuf)

    pltpu.sync_copy(cnts_hbm.at[wid], cbuf)
    nb = jnp.max(cbuf[...])

    def batch(b, carry):
        pltpu.sync_copy(lsrc_hbm.at[wid, pl.ds(b * _CAP, _CAP)], bsrc)
        pltpu.sync_copy(ldst_hbm.at[wid, pl.ds(b * _CAP, _CAP)], bdst)
        _accumulate(t_hbm, aggbuf, rowbuf, bsrc, bdst, sem)
        return carry

    lax.fori_loop(0, nb, batch, 0)

    pltpu.sync_copy(aggbuf.at[pl.ds(0, _RPT)], out_hbm.at[pl.ds(lo, _RPT)])


def _sc_compiler_params():
    cp = pltpu.CompilerParams()
    if "needs_layout_passes" in pltpu.CompilerParams.__dataclass_fields__:
        cp = dataclasses.replace(cp, needs_layout_passes=False)
    return cp


def _segmax_scan(t, src, dst):
    k = pl.kernel(
        _segmax_scan_body,
        out_type=(
            jax.ShapeDtypeStruct((_NPAD, _DW), jnp.int32),
            jax.ShapeDtypeStruct((_NW, _ESLOT), jnp.int32),
            jax.ShapeDtypeStruct((_NW, _ESLOT), jnp.int32),
            jax.ShapeDtypeStruct((_NW, 16), jnp.int32),
        ),
        compiler_params=_sc_compiler_params(),
        mesh=plsc.VectorSubcoreMesh(core_axis_name="c", subcore_axis_name="s"),
        scratch_types=[
            pltpu.VMEM((_RPT + 1, _DW), jnp.int32),
            pltpu.VMEM((_CAP, _DW), jnp.int32),
            pltpu.VMEM((_CAP,), jnp.int32),
            pltpu.VMEM((_CAP,), jnp.int32),
            pltpu.VMEM((_SECH,), jnp.int32),
            pltpu.VMEM((_SECH,), jnp.int32),
            pltpu.VMEM((16,), jnp.int32),
            pltpu.SemaphoreType.DMA,
        ],
    )
    return k(t, src, dst)


def _segmax_replay(t, lsrc, ldst, cnts):
    k = pl.kernel(
        _segmax_replay_body,
        out_type=jax.ShapeDtypeStruct((_NPAD, _DW), jnp.int32),
        compiler_params=_sc_compiler_params(),
        mesh=plsc.VectorSubcoreMesh(core_axis_name="c", subcore_axis_name="s"),
        scratch_types=[
            pltpu.VMEM((_RPT + 1, _DW), jnp.int32),
            pltpu.VMEM((_CAP, _DW), jnp.int32),
            pltpu.VMEM((_CAP,), jnp.int32),
            pltpu.VMEM((_CAP,), jnp.int32),
            pltpu.VMEM((16,), jnp.int32),
            pltpu.SemaphoreType.DMA,
        ],
    )
    return k(t, lsrc, ldst, cnts)


def _layer0_update(f0, agg0, w_bot):
    def body(f0_ref, agg_ref, w_ref, u_ref, ps_ref, pss_ref):
        y = f0_ref[...] + jnp.dot(agg_ref[...].astype(jnp.float32),
                                  w_ref[...],
                                  preferred_element_type=jnp.float32)
        u = jnp.maximum(y, 0.0)
        u_ref[...] = u
        ps_ref[0, ...] = jnp.sum(u, axis=0, keepdims=True)
        pss_ref[0, ...] = jnp.sum(u * u, axis=0, keepdims=True)

    return pl.pallas_call(
        body,
        grid=(_NBLK,),
        in_specs=[
            pl.BlockSpec((_ROWS, _D), lambda i: (i, 0)),
            pl.BlockSpec((_ROWS, _D), lambda i: (i, 0)),
            pl.BlockSpec((_D, _D), lambda i: (0, 0)),
        ],
        out_specs=[
            pl.BlockSpec((_ROWS, _D), lambda i: (i, 0)),
            pl.BlockSpec((1, 1, _D), lambda i: (i, 0, 0)),
            pl.BlockSpec((1, 1, _D), lambda i: (i, 0, 0)),
        ],
        out_shape=[
            jax.ShapeDtypeStruct((_N, _D), jnp.float32),
            jax.ShapeDtypeStruct((_NBLK, 1, _D), jnp.float32),
            jax.ShapeDtypeStruct((_NBLK, 1, _D), jnp.float32),
        ],
    )(f0, agg0, w_bot)


def _bn_norm_t1(u, ps, pss, gamma, beta, Wp1, bp1):
    def body(u_ref, ps_ref, pss_ref, g_ref, be_ref, w_ref, b_ref,
             h_ref, t1_ref):
        mean = jnp.sum(ps_ref[...], axis=0) / _N
        var = jnp.sum(pss_ref[...], axis=0) / _N - mean * mean
        inv = jax.lax.rsqrt(var + 1e-5)
        hb = (u_ref[...] - mean) * inv * g_ref[...] + be_ref[...]
        norm = jnp.sqrt(jnp.sum(hb * hb, axis=1, keepdims=True))
        hn = hb / (norm + 1e-6)
        h_ref[...] = hn
        t1_ref[...] = jnp.maximum(
            jnp.dot(hn, w_ref[...], preferred_element_type=jnp.float32)
            + b_ref[...], 0.0).astype(jnp.bfloat16)

    return pl.pallas_call(
        body,
        grid=(_NBLK,),
        in_specs=[
            pl.BlockSpec((_ROWS, _D), lambda i: (i, 0)),
            pl.BlockSpec((_NBLK, 1, _D), lambda i: (0, 0, 0)),
            pl.BlockSpec((_NBLK, 1, _D), lambda i: (0, 0, 0)),
            pl.BlockSpec((1, _D), lambda i: (0, 0)),
            pl.BlockSpec((1, _D), lambda i: (0, 0)),
            pl.BlockSpec((_D, _D), lambda i: (0, 0)),
            pl.BlockSpec((1, _D), lambda i: (0, 0)),
        ],
        out_specs=[
            pl.BlockSpec((_ROWS, _D), lambda i: (i, 0)),
            pl.BlockSpec((_ROWS, _D), lambda i: (i, 0)),
        ],
        out_shape=[
            jax.ShapeDtypeStruct((_N, _D), jnp.float32),
            jax.ShapeDtypeStruct((_N, _D), jnp.bfloat16),
        ],
    )(u, ps, pss, gamma.reshape(1, _D), beta.reshape(1, _D), Wp1,
      bp1.reshape(1, _D))


def _final(h1, agg1, w_bot):
    def body(h1_ref, agg_ref, w_ref, o_ref):
        o_ref[...] = h1_ref[...] + jnp.dot(
            agg_ref[...].astype(jnp.float32), w_ref[...],
            preferred_element_type=jnp.float32)

    return pl.pallas_call(
        body,
        grid=(_NBLK,),
        in_specs=[
            pl.BlockSpec((_ROWS, _D), lambda i: (i, 0)),
            pl.BlockSpec((_ROWS, _D), lambda i: (i, 0)),
            pl.BlockSpec((_D, _D), lambda i: (0, 0)),
        ],
        out_specs=pl.BlockSpec((_ROWS, _D), lambda i: (i, 0)),
        out_shape=jax.ShapeDtypeStruct((_N, _D), jnp.float32),
    )(h1, agg1, w_bot)


def _pack(t):
    return lax.bitcast_convert_type(t.reshape(_N, _DW, 2), jnp.int32)


def _unpack(a):
    return lax.bitcast_convert_type(a, jnp.bfloat16).reshape(-1, _D)


def kernel(features, edge_index, Wp0, bp0, Wp1, bp1, Wfc0, bfc0, Wfc1, bfc1,
           gamma0, beta0):
    src = edge_index[0]
    dst = edge_index[1]
    t0 = _linear(features, Wp0, bp0, relu=True, out_dtype=jnp.bfloat16)
    f0 = _linear(features, Wfc0[:_D], bfc0, relu=False)
    agg0, lsrc, ldst, cnts = _segmax_scan(_pack(t0), src, dst)
    agg0 = _unpack(agg0)[:_N]
    u, ps, pss = _layer0_update(f0, agg0, Wfc0[_D:])
    h, t1 = _bn_norm_t1(u, ps, pss, gamma0, beta0, Wp1, bp1)
    h1 = _linear(h, Wfc1[:_D], bfc1, relu=False)
    agg1 = _unpack(_segmax_replay(_pack(t1), lsrc, ldst, cnts))[:_N]
    out = _final(h1, agg1, Wfc1[_D:])
    return out
